# Initial kernel scaffold; baseline (speedup 1.0000x reference)
import jax
import jax.numpy as jnp
from jax.experimental import pallas as pl


def kernel(lut, states):
    raise NotImplementedError("write your pallas kernel here")


# speedup vs baseline: 1.9537x; 1.9537x over previous
import functools

import jax
import jax.numpy as jnp
from jax import lax
from jax.experimental import pallas as pl
from jax.experimental.pallas import tpu as pltpu
from jax.experimental.pallas import tpu_sc as plsc

_NUM_CORES = 2
_NUM_SUBCORES = 16
_NUM_WORKERS = _NUM_CORES * _NUM_SUBCORES
_LANES = 16

_VALUES = 64
_STATES = 8192
_NUM_IDX = 16 * 4096
_ROWS_PER_WORKER = _VALUES // _NUM_WORKERS
_CHUNK = 4096
_UNROLL = 8


def _gather_body(lut_hbm, idx_hbm, out_hbm, idx_v, rows_v, out_buf):
  wid = lax.axis_index("s") * _NUM_CORES + lax.axis_index("c")
  r0 = wid * _ROWS_PER_WORKER

  pltpu.sync_copy(lut_hbm.at[pl.ds(r0 * _STATES, _ROWS_PER_WORKER * _STATES)],
                  rows_v)
  pltpu.sync_copy(idx_hbm, idx_v)

  n_chunks = _NUM_IDX // _CHUNK
  steps = _CHUNK // (_LANES * _UNROLL)

  for r in range(_ROWS_PER_WORKER):
    row_off = jnp.int32(r * _STATES)

    def chunk_body(c, _, row_off=row_off, r=r):
      c_base = c * _CHUNK

      def body(i, _):
        base = i * (_LANES * _UNROLL)
        for u in range(_UNROLL):
          off = base + u * _LANES
          iv = idx_v[pl.ds(c_base + off, _LANES)] + row_off
          out_buf[pl.ds(off, _LANES)] = plsc.load_gather(rows_v, [iv])
        return 0

      lax.fori_loop(0, steps, body, 0, unroll=False)
      out_start = (r0 + r) * _NUM_IDX + c_base
      pltpu.sync_copy(out_buf, out_hbm.at[pl.ds(out_start, _CHUNK)])
      return 0

    lax.fori_loop(0, n_chunks, chunk_body, 0, unroll=False)


@jax.jit
def _reconstruct(lut, states):
  idx = states.reshape(-1)
  lut_flat = lut.reshape(-1)
  mesh = plsc.VectorSubcoreMesh(
      core_axis_name="c", subcore_axis_name="s",
      num_cores=_NUM_CORES, num_subcores=_NUM_SUBCORES)
  out_flat = pl.kernel(
      _gather_body,
      out_type=jax.ShapeDtypeStruct((_VALUES * _NUM_IDX,), jnp.float32),
      mesh=mesh,
      compiler_params=pltpu.CompilerParams(needs_layout_passes=False),
      scratch_types=[
          pltpu.VMEM((_NUM_IDX,), jnp.int32),
          pltpu.VMEM((_ROWS_PER_WORKER * _STATES,), jnp.float32),
          pltpu.VMEM((_CHUNK,), jnp.float32),
      ],
  )(lut_flat, idx)
  return out_flat.reshape(_VALUES, *states.shape)


def kernel(lut, states):
  return _reconstruct(lut, states)


# speedup vs baseline: 2.5525x; 1.3065x over previous
import jax
import jax.numpy as jnp
from jax import lax
from jax.experimental import pallas as pl
from jax.experimental.pallas import tpu as pltpu
from jax.experimental.pallas import tpu_sc as plsc

_NUM_CORES = 2
_NUM_SUBCORES = 16
_NUM_WORKERS = _NUM_CORES * _NUM_SUBCORES
_LANES = 16

_VALUES = 64
_STATES = 8192
_NUM_IDX = 16 * 4096
_ROWS_PER_WORKER = _VALUES // _NUM_WORKERS
_CHUNK = 4096
_UNROLL = 8
_N_CHUNKS = _NUM_IDX // _CHUNK


def _gather_body(lut_hbm, idx_hbm, out_hbm, idx_v, rows_v,
                 ob00, ob01, ob10, ob11, sem_idx, sem_rows,
                 s00, s01, s10, s11):
  wid = lax.axis_index("s") * _NUM_CORES + lax.axis_index("c")
  r0 = wid * _ROWS_PER_WORKER

  out_bufs = ((ob00, ob01), (ob10, ob11))
  sems = ((s00, s01), (s10, s11))

  rows_d = pltpu.async_copy(
      lut_hbm.at[pl.ds(r0 * _STATES, _ROWS_PER_WORKER * _STATES)], rows_v,
      sem_rows)
  idx_d = pltpu.async_copy(idx_hbm, idx_v, sem_idx)
  rows_d.wait()
  idx_d.wait()

  steps = _CHUNK // (_LANES * _UNROLL)

  def super_step(s, _):
    for p in range(2):
      c = s * 2 + p
      c_base = c * _CHUNK

      @pl.when(s > 0)
      def _():
        for r in range(2):
          prev = (r0 + r) * _NUM_IDX + (c - 2) * _CHUNK
          pltpu.make_async_copy(
              out_bufs[p][r], out_hbm.at[pl.ds(prev, _CHUNK)],
              sems[p][r]).wait()

      def gbody(i, _):
        base = i * (_LANES * _UNROLL)
        for u in range(_UNROLL):
          off = base + u * _LANES
          iv = idx_v[pl.ds(c_base + off, _LANES)]
          out_bufs[p][0][pl.ds(off, _LANES)] = plsc.load_gather(rows_v, [iv])
          out_bufs[p][1][pl.ds(off, _LANES)] = plsc.load_gather(
              rows_v, [iv + _STATES])
        return 0

      lax.fori_loop(0, steps, gbody, 0, unroll=False)

      for r in range(2):
        pltpu.async_copy(
            out_bufs[p][r],
            out_hbm.at[pl.ds((r0 + r) * _NUM_IDX + c_base, _CHUNK)],
            sems[p][r])
    return 0

  lax.fori_loop(0, _N_CHUNKS // 2, super_step, 0, unroll=False)

  for p in range(2):
    c = _N_CHUNKS - 2 + p
    for r in range(2):
      last = (r0 + r) * _NUM_IDX + c * _CHUNK
      pltpu.make_async_copy(
          out_bufs[p][r], out_hbm.at[pl.ds(last, _CHUNK)], sems[p][r]).wait()


@jax.jit
def _reconstruct(lut, states):
  idx = states.reshape(-1)
  lut_flat = lut.reshape(-1)
  mesh = plsc.VectorSubcoreMesh(
      core_axis_name="c", subcore_axis_name="s",
      num_cores=_NUM_CORES, num_subcores=_NUM_SUBCORES)
  out_flat = pl.kernel(
      _gather_body,
      out_type=jax.ShapeDtypeStruct((_VALUES * _NUM_IDX,), jnp.float32),
      mesh=mesh,
      compiler_params=pltpu.CompilerParams(needs_layout_passes=False),
      scratch_types=[
          pltpu.VMEM((_NUM_IDX,), jnp.int32),
          pltpu.VMEM((_ROWS_PER_WORKER * _STATES,), jnp.float32),
          pltpu.VMEM((_CHUNK,), jnp.float32),
          pltpu.VMEM((_CHUNK,), jnp.float32),
          pltpu.VMEM((_CHUNK,), jnp.float32),
          pltpu.VMEM((_CHUNK,), jnp.float32),
          pltpu.SemaphoreType.DMA,
          pltpu.SemaphoreType.DMA,
          pltpu.SemaphoreType.DMA,
          pltpu.SemaphoreType.DMA,
          pltpu.SemaphoreType.DMA,
          pltpu.SemaphoreType.DMA,
      ],
  )(lut_flat, idx)
  return out_flat.reshape(_VALUES, *states.shape)


def kernel(lut, states):
  return _reconstruct(lut, states)


# speedup vs baseline: 4.5582x; 1.7858x over previous
import jax
import jax.numpy as jnp
from jax import lax
from jax.experimental import pallas as pl
from jax.experimental.pallas import tpu as pltpu
from jax.experimental.pallas import tpu_sc as plsc

_NUM_CORES = 2
_NUM_SUBCORES = 16
_NUM_WORKERS = _NUM_CORES * _NUM_SUBCORES
_LANES = 16

_VALUES = 64
_STATES = 8192
_NUM_IDX = 16 * 4096
_ROWS_PER_WORKER = _VALUES // _NUM_WORKERS
_CHUNK = 4096
_UNROLL = 8
_N_CHUNKS = _NUM_IDX // _CHUNK


def _gather_body(lut_hbm, idx_hbm, out_hbm, idx_v, rows_v,
                 ob00, ob01, ob10, ob11, sem_idx, sem_rows,
                 s00, s01, s10, s11):
  wid = lax.axis_index("s") * _NUM_CORES + lax.axis_index("c")
  r0 = wid * _ROWS_PER_WORKER

  out_bufs = ((ob00, ob01), (ob10, ob11))
  sems = ((s00, s01), (s10, s11))

  rows_d = pltpu.async_copy(
      lut_hbm.at[pl.ds(r0 * _STATES, _ROWS_PER_WORKER * _STATES)], rows_v,
      sem_rows)
  idx_d = pltpu.async_copy(idx_hbm, idx_v, sem_idx)
  rows_d.wait()
  idx_d.wait()

  def super_step(s, _):
    for p in range(2):
      c = s * 2 + p
      c_base = c * _CHUNK

      @pl.when(s > 0)
      def _():
        for r in range(2):
          prev = (r0 + r) * _NUM_IDX + (c - 2) * _CHUNK
          pltpu.make_async_copy(
              out_bufs[p][r], out_hbm.at[pl.ds(prev, _CHUNK)],
              sems[p][r]).wait()

      @plsc.parallel_loop(0, _CHUNK // _LANES, unroll=_UNROLL)
      def gbody(i):
        off = i * _LANES
        iv = idx_v[pl.ds(c_base + off, _LANES)]
        out_bufs[p][0][pl.ds(off, _LANES)] = plsc.load_gather(rows_v, [iv])
        out_bufs[p][1][pl.ds(off, _LANES)] = plsc.load_gather(
            rows_v, [iv + _STATES])

      for r in range(2):
        pltpu.async_copy(
            out_bufs[p][r],
            out_hbm.at[pl.ds((r0 + r) * _NUM_IDX + c_base, _CHUNK)],
            sems[p][r])
    return 0

  lax.fori_loop(0, _N_CHUNKS // 2, super_step, 0, unroll=False)

  for p in range(2):
    c = _N_CHUNKS - 2 + p
    for r in range(2):
      last = (r0 + r) * _NUM_IDX + c * _CHUNK
      pltpu.make_async_copy(
          out_bufs[p][r], out_hbm.at[pl.ds(last, _CHUNK)], sems[p][r]).wait()


@jax.jit
def _reconstruct(lut, states):
  idx = states.reshape(-1)
  lut_flat = lut.reshape(-1)
  mesh = plsc.VectorSubcoreMesh(
      core_axis_name="c", subcore_axis_name="s",
      num_cores=_NUM_CORES, num_subcores=_NUM_SUBCORES)
  out_flat = pl.kernel(
      _gather_body,
      out_type=jax.ShapeDtypeStruct((_VALUES * _NUM_IDX,), jnp.float32),
      mesh=mesh,
      compiler_params=pltpu.CompilerParams(needs_layout_passes=False),
      scratch_types=[
          pltpu.VMEM((_NUM_IDX,), jnp.int32),
          pltpu.VMEM((_ROWS_PER_WORKER * _STATES,), jnp.float32),
          pltpu.VMEM((_CHUNK,), jnp.float32),
          pltpu.VMEM((_CHUNK,), jnp.float32),
          pltpu.VMEM((_CHUNK,), jnp.float32),
          pltpu.VMEM((_CHUNK,), jnp.float32),
          pltpu.SemaphoreType.DMA,
          pltpu.SemaphoreType.DMA,
          pltpu.SemaphoreType.DMA,
          pltpu.SemaphoreType.DMA,
          pltpu.SemaphoreType.DMA,
          pltpu.SemaphoreType.DMA,
      ],
  )(lut_flat, idx)
  return out_flat.reshape(_VALUES, *states.shape)


def kernel(lut, states):
  return _reconstruct(lut, states)


# speedup vs baseline: 6.8845x; 1.5104x over previous
import jax
import jax.numpy as jnp
from jax import lax
from jax.experimental import pallas as pl
from jax.experimental.pallas import tpu as pltpu
from jax.experimental.pallas import tpu_sc as plsc

_NUM_CORES = 2
_NUM_SUBCORES = 16
_NUM_WORKERS = _NUM_CORES * _NUM_SUBCORES
_LANES = 16

_VALUES = 64
_STATES = 8192
_BATCH = 16
_TOKENS = 4096
_ROWS_PER_WORKER = _VALUES // _NUM_WORKERS
_UNROLL = 8


def _gather_body(lut_hbm, idx_hbm, out_hbm, idx_v, rows_v,
                 ob00, ob01, ob10, ob11, sem_idx, sem_rows,
                 s00, s01, s10, s11):
  wid = lax.axis_index("s") * _NUM_CORES + lax.axis_index("c")
  r0 = wid * _ROWS_PER_WORKER

  out_bufs = ((ob00, ob01), (ob10, ob11))
  sems = ((s00, s01), (s10, s11))

  rows_d = pltpu.async_copy(lut_hbm.at[pl.ds(r0, _ROWS_PER_WORKER), :],
                            rows_v, sem_rows)
  idx_d = pltpu.async_copy(idx_hbm, idx_v, sem_idx)
  rows_d.wait()
  idx_d.wait()

  row_sel = [jnp.full((_LANES,), r, jnp.int32)
             for r in range(_ROWS_PER_WORKER)]

  def super_step(s, _):
    for p in range(2):
      b = s * 2 + p

      @pl.when(s > 0)
      def _():
        for r in range(2):
          pltpu.make_async_copy(
              out_bufs[p][r], out_hbm.at[r0 + r, b - 2, :],
              sems[p][r]).wait()

      @plsc.parallel_loop(0, _TOKENS // _LANES, unroll=_UNROLL)
      def gbody(i):
        off = i * _LANES
        iv = idx_v[b, pl.ds(off, _LANES)]
        out_bufs[p][0][pl.ds(off, _LANES)] = plsc.load_gather(
            rows_v, [row_sel[0], iv])
        out_bufs[p][1][pl.ds(off, _LANES)] = plsc.load_gather(
            rows_v, [row_sel[1], iv])

      for r in range(2):
        pltpu.async_copy(out_bufs[p][r], out_hbm.at[r0 + r, b, :],
                         sems[p][r])
    return 0

  lax.fori_loop(0, _BATCH // 2, super_step, 0, unroll=False)

  for p in range(2):
    b = _BATCH - 2 + p
    for r in range(2):
      pltpu.make_async_copy(
          out_bufs[p][r], out_hbm.at[r0 + r, b, :], sems[p][r]).wait()


@jax.jit
def _reconstruct(lut, states):
  mesh = plsc.VectorSubcoreMesh(
      core_axis_name="c", subcore_axis_name="s",
      num_cores=_NUM_CORES, num_subcores=_NUM_SUBCORES)
  return pl.kernel(
      _gather_body,
      out_type=jax.ShapeDtypeStruct((_VALUES, _BATCH, _TOKENS), jnp.float32),
      mesh=mesh,
      compiler_params=pltpu.CompilerParams(needs_layout_passes=False),
      scratch_types=[
          pltpu.VMEM((_BATCH, _TOKENS), jnp.int32),
          pltpu.VMEM((_ROWS_PER_WORKER, _STATES), jnp.float32),
          pltpu.VMEM((_TOKENS,), jnp.float32),
          pltpu.VMEM((_TOKENS,), jnp.float32),
          pltpu.VMEM((_TOKENS,), jnp.float32),
          pltpu.VMEM((_TOKENS,), jnp.float32),
          pltpu.SemaphoreType.DMA,
          pltpu.SemaphoreType.DMA,
          pltpu.SemaphoreType.DMA,
          pltpu.SemaphoreType.DMA,
          pltpu.SemaphoreType.DMA,
          pltpu.SemaphoreType.DMA,
      ],
  )(lut, states)


def kernel(lut, states):
  return _reconstruct(lut, states)
